# baseline (speedup 1.0000x reference)
import functools

import jax
import jax.numpy as jnp
from jax import lax
from jax.experimental import pallas as pl
from jax.experimental.pallas import tpu as pltpu
from jax.experimental.pallas import tpu_sc as plsc

NC = 2
NS = 16
NW = NC * NS
LANES = 16
CHUNK = 256
DPAD = 128
G = 64


def _mesh():
    return plsc.VectorSubcoreMesh(core_axis_name="c", subcore_axis_name="s")


def _sc_degree(n_pad, ep):
    ept = ep // NW
    rpt = n_pad // NS

    @functools.partial(
        pl.kernel,
        out_type=jax.ShapeDtypeStruct((NC, n_pad), jnp.float32),
        mesh=_mesh(),
        scratch_types=[
            pltpu.VMEM((ept,), jnp.int32),
            pltpu.VMEM((n_pad,), jnp.float32),
            pltpu.VMEM((rpt,), jnp.float32),
            pltpu.VMEM((rpt,), jnp.float32),
            pltpu.VMEM_SHARED((NS, n_pad), jnp.float32),
        ],
        compiler_params=pltpu.CompilerParams(needs_layout_passes=False),
    )
    def k(dst_hbm, out_hbm, dstbuf, hist, asum, tmp, stage):
        cid = lax.axis_index("c")
        sid = lax.axis_index("s")
        wid = sid * NC + cid

        def zero(i, carry):
            hist[pl.ds(i * LANES, LANES)] = jnp.zeros((LANES,), jnp.float32)
            return carry

        lax.fori_loop(0, n_pad // LANES, zero, 0)
        pltpu.sync_copy(dst_hbm.at[pl.ds(wid * ept, ept)], dstbuf)

        ones16 = jnp.ones((LANES,), jnp.float32)

        def scat(i, carry):
            idxv = dstbuf[pl.ds(i * LANES, LANES)]
            plsc.addupdate_scatter(hist, [idxv], ones16)
            return carry

        lax.fori_loop(0, ept // LANES, scat, 0)

        pltpu.sync_copy(hist, stage.at[sid])
        plsc.subcore_barrier()

        base = sid * rpt
        pltpu.sync_copy(stage.at[0, pl.ds(base, rpt)], asum)
        for t in range(1, NS):
            pltpu.sync_copy(stage.at[t, pl.ds(base, rpt)], tmp)

            def acc(j, carry):
                sl = pl.ds(j * LANES, LANES)
                asum[sl] = asum[sl] + tmp[sl]
                return carry

            lax.fori_loop(0, rpt // LANES, acc, 0)
        pltpu.sync_copy(asum, out_hbm.at[cid, pl.ds(base, rpt)])

    return k


def _sc_scatter(n_pad, ep):
    cpt = ep // (NW * CHUNK)
    rpt = n_pad // NS

    @functools.partial(
        pl.kernel,
        out_type=jax.ShapeDtypeStruct((NC, n_pad, DPAD), jnp.float32),
        mesh=_mesh(),
        scratch_types=[
            pltpu.VMEM((CHUNK,), jnp.int32),
            pltpu.VMEM((CHUNK,), jnp.int32),
            pltpu.VMEM((CHUNK, DPAD), jnp.float32),
            pltpu.VMEM_SHARED((n_pad, DPAD), jnp.float32),
            pltpu.SemaphoreType.DMA,
        ],
        compiler_params=pltpu.CompilerParams(needs_layout_passes=False),
    )
    def k(h_hbm, src_hbm, dst_hbm, out_hbm, idx_s, idx_d, rows, acc, sem):
        cid = lax.axis_index("c")
        sid = lax.axis_index("s")
        wid = sid * NC + cid

        def zr(r, carry):
            for j in range(DPAD // LANES):
                rows[r, pl.ds(j * LANES, LANES)] = jnp.zeros((LANES,),
                                                             jnp.float32)
            return carry

        lax.fori_loop(0, 128, zr, 0)
        for z in range(rpt // 128):
            pltpu.sync_copy(rows.at[pl.ds(0, 128)],
                            acc.at[pl.ds(sid * rpt + z * 128, 128)])
        plsc.subcore_barrier()

        def body(c, carry):
            base = (wid * cpt + c) * CHUNK
            pltpu.sync_copy(src_hbm.at[pl.ds(base, CHUNK)], idx_s)
            pltpu.sync_copy(dst_hbm.at[pl.ds(base, CHUNK)], idx_d)
            pltpu.async_copy(h_hbm.at[idx_s], rows, sem).wait()
            pltpu.sync_copy(rows, acc.at[idx_d], add=True)
            return carry

        lax.fori_loop(0, cpt, body, 0)
        plsc.subcore_barrier()
        pltpu.sync_copy(acc.at[pl.ds(sid * rpt, rpt)],
                        out_hbm.at[cid, pl.ds(sid * rpt, rpt)])

    return k


def _tc_first(n, d1):
    def body(d0_ref, dg1_ref, x_ref, w_ref, h_ref, dis_ref):
        dis = lax.rsqrt(d0_ref[...] + dg1_ref[...])
        dis_ref[...] = dis
        h = jnp.dot(x_ref[...], w_ref[...],
                    preferred_element_type=jnp.float32) * dis
        h_ref[:, 0:d1] = h
        h_ref[:, d1:DPAD] = jnp.zeros((n, DPAD - d1), jnp.float32)
    return body


def _tc_mid(n, din):
    def body(s_ref, dis_ref, b_ref, w_ref, out_ref):
        dis = dis_ref[...]
        s = s_ref[0, :n, 0:din] + s_ref[1, :n, 0:din]
        h = jnp.maximum(s * dis + b_ref[...], 0.0)
        out_ref[...] = jnp.dot(h, w_ref[...],
                               preferred_element_type=jnp.float32) * dis
    return body


def _tc_head(n):
    def body(s_ref, dis_ref, b_ref, batch_ref, wf1_ref, bf1_ref,
             wf2_ref, bf2_ref, out_ref):
        s = s_ref[0, :n, :] + s_ref[1, :n, :]
        h3 = jnp.maximum(s * dis_ref[...] + b_ref[...], 0.0)
        gid = lax.broadcasted_iota(jnp.int32, (G, 1), 0)
        oh = (gid == batch_ref[...]).astype(jnp.float32)
        sums = jnp.dot(oh, h3, preferred_element_type=jnp.float32)
        cnt = jnp.sum(oh, axis=1, keepdims=True)
        pooled = sums / jnp.maximum(cnt, 1.0)
        z = jnp.maximum(jnp.dot(pooled, wf1_ref[...],
                                preferred_element_type=jnp.float32)
                        + bf1_ref[...], 0.0)
        out_ref[...] = jnp.dot(z, wf2_ref[...],
                               preferred_element_type=jnp.float32) + bf2_ref[...]
    return body


def kernel(x, edge_index, batch, W1, b1, W2, b2, W3, b3, Wf1, bf1, Wf2, bf2):
    n, feat = x.shape
    e = edge_index.shape[1]
    et = e + n
    gran = NW * CHUNK
    ep = -(-et // gran) * gran
    cpt = ep // (NW * CHUNK)
    n_pad = -(-(n + 1) // (NS * 128)) * (NS * 128)

    f32 = jnp.float32
    loop = jnp.arange(n, dtype=jnp.int32)
    pad_s = jnp.zeros((ep - et,), jnp.int32)
    pad_d = jnp.full((ep - et,), n, jnp.int32)
    src = jnp.concatenate([edge_index[0], loop, pad_s,
                           jnp.zeros((CHUNK,), jnp.int32)])
    dst = jnp.concatenate([edge_index[1], loop, pad_d,
                           jnp.full((CHUNK,), n, jnp.int32)])

    degp = _sc_degree(n_pad, ep)(dst)
    deg0 = degp[0, :n].reshape(n, 1)
    deg1 = degp[1, :n].reshape(n, 1)

    d1 = W1.shape[1]
    h1, dis = pl.pallas_call(
        _tc_first(n, d1),
        out_shape=(jax.ShapeDtypeStruct((n, DPAD), f32),
                   jax.ShapeDtypeStruct((n, 1), f32)),
    )(deg0, deg1, x, W1)

    s1 = _sc_scatter(n_pad, ep)(h1, src, dst)

    d2 = W2.shape[1]
    h2 = pl.pallas_call(
        _tc_mid(n, d1), out_shape=jax.ShapeDtypeStruct((n, d2), f32),
    )(s1, dis, b1.reshape(1, -1), W2)

    s2 = _sc_scatter(n_pad, ep)(h2, src, dst)

    d3 = W3.shape[1]
    h3 = pl.pallas_call(
        _tc_mid(n, d2), out_shape=jax.ShapeDtypeStruct((n, d3), f32),
    )(s2, dis, b2.reshape(1, -1), W3)

    s3 = _sc_scatter(n_pad, ep)(h3, src, dst)

    out = pl.pallas_call(
        _tc_head(n), out_shape=jax.ShapeDtypeStruct((G, 1), f32),
    )(s3, dis, b3.reshape(1, -1), batch, Wf1, bf1.reshape(1, -1),
      Wf2, bf2.reshape(1, -1))
    return out


# speedup vs baseline: 1.5257x; 1.5257x over previous
import functools

import jax
import jax.numpy as jnp
from jax import lax
from jax.experimental import pallas as pl
from jax.experimental.pallas import tpu as pltpu
from jax.experimental.pallas import tpu_sc as plsc

NC = 2
NS = 16
NW = NC * NS
LANES = 16
CHUNK = 128
DPAD = 128
G = 64


def _mesh():
    return plsc.VectorSubcoreMesh(core_axis_name="c", subcore_axis_name="s")


def _sc_degree(n_pad, ep):
    ept = ep // NW
    rpt = n_pad // NS

    @functools.partial(
        pl.kernel,
        out_type=jax.ShapeDtypeStruct((NC, n_pad), jnp.float32),
        mesh=_mesh(),
        scratch_types=[
            pltpu.VMEM((ept,), jnp.int32),
            pltpu.VMEM((n_pad,), jnp.float32),
            pltpu.VMEM((rpt,), jnp.float32),
            pltpu.VMEM((rpt,), jnp.float32),
            pltpu.VMEM_SHARED((NS, n_pad), jnp.float32),
        ],
        compiler_params=pltpu.CompilerParams(needs_layout_passes=False),
    )
    def k(dst_hbm, out_hbm, dstbuf, hist, asum, tmp, stage):
        cid = lax.axis_index("c")
        sid = lax.axis_index("s")
        wid = sid * NC + cid

        def zero(i, carry):
            hist[pl.ds(i * LANES, LANES)] = jnp.zeros((LANES,), jnp.float32)
            return carry

        lax.fori_loop(0, n_pad // LANES, zero, 0)
        pltpu.sync_copy(dst_hbm.at[pl.ds(wid * ept, ept)], dstbuf)

        ones16 = jnp.ones((LANES,), jnp.float32)

        def scat(i, carry):
            idxv = dstbuf[pl.ds(i * LANES, LANES)]
            plsc.addupdate_scatter(hist, [idxv], ones16)
            return carry

        lax.fori_loop(0, ept // LANES, scat, 0)

        pltpu.sync_copy(hist, stage.at[sid])
        plsc.subcore_barrier()

        base = sid * rpt
        pltpu.sync_copy(stage.at[0, pl.ds(base, rpt)], asum)
        for t in range(1, NS):
            pltpu.sync_copy(stage.at[t, pl.ds(base, rpt)], tmp)

            def acc(j, carry):
                sl = pl.ds(j * LANES, LANES)
                asum[sl] = asum[sl] + tmp[sl]
                return carry

            lax.fori_loop(0, rpt // LANES, acc, 0)
        pltpu.sync_copy(asum, out_hbm.at[cid, pl.ds(base, rpt)])

    return k


def _sc_scatter(n_pad, ep, d):
    cpt = ep // (NW * CHUNK)
    rpt = n_pad // NS

    @functools.partial(
        pl.kernel,
        out_type=jax.ShapeDtypeStruct((NC, n_pad, d), jnp.float32),
        mesh=_mesh(),
        scratch_types=[
            pltpu.VMEM((CHUNK,), jnp.int32),
            pltpu.VMEM((CHUNK,), jnp.int32),
            pltpu.VMEM((CHUNK, d), jnp.float32),
            pltpu.VMEM_SHARED((n_pad, d), jnp.float32),
            pltpu.SemaphoreType.DMA,
        ],
        compiler_params=pltpu.CompilerParams(needs_layout_passes=False,
                                             use_tc_tiling_on_sc=False),
    )
    def k(h_hbm, src_hbm, dst_hbm, out_hbm, idx_s, idx_d, rows, acc, sem):
        cid = lax.axis_index("c")
        sid = lax.axis_index("s")
        wid = sid * NC + cid

        def zr(r, carry):
            for j in range(d // LANES):
                rows[r, pl.ds(j * LANES, LANES)] = jnp.zeros((LANES,),
                                                             jnp.float32)
            return carry

        lax.fori_loop(0, 128, zr, 0)
        for z in range(rpt // 128):
            pltpu.sync_copy(rows.at[pl.ds(0, 128)],
                            acc.at[pl.ds(sid * rpt + z * 128, 128)])
        plsc.subcore_barrier()

        def body(c, carry):
            base = (wid * cpt + c) * CHUNK
            pltpu.sync_copy(src_hbm.at[pl.ds(base, CHUNK)], idx_s)
            pltpu.sync_copy(dst_hbm.at[pl.ds(base, CHUNK)], idx_d)
            pltpu.async_copy(h_hbm.at[idx_s], rows, sem).wait()
            pltpu.sync_copy(rows, acc.at[idx_d], add=True)
            return carry

        lax.fori_loop(0, cpt, body, 0)
        plsc.subcore_barrier()
        pltpu.sync_copy(acc.at[pl.ds(sid * rpt, rpt)],
                        out_hbm.at[cid, pl.ds(sid * rpt, rpt)])

    return k


def _tc_first(n):
    def body(d0_ref, dg1_ref, x_ref, w_ref, h_ref, dis_ref):
        dis = lax.rsqrt(d0_ref[...] + dg1_ref[...])
        dis_ref[...] = dis
        h_ref[...] = jnp.dot(x_ref[...], w_ref[...],
                             preferred_element_type=jnp.float32) * dis
    return body


def _tc_mid(n, din):
    def body(s_ref, dis_ref, b_ref, w_ref, out_ref):
        dis = dis_ref[...]
        s = s_ref[0, :n, :] + s_ref[1, :n, :]
        h = jnp.maximum(s * dis + b_ref[...], 0.0)
        out_ref[...] = jnp.dot(h, w_ref[...],
                               preferred_element_type=jnp.float32) * dis
    return body


def _tc_head(n):
    def body(s_ref, dis_ref, b_ref, batch_ref, wf1_ref, bf1_ref,
             wf2_ref, bf2_ref, out_ref):
        s = s_ref[0, :n, :] + s_ref[1, :n, :]
        h3 = jnp.maximum(s * dis_ref[...] + b_ref[...], 0.0)
        gid = lax.broadcasted_iota(jnp.int32, (G, 1), 0)
        oh = (gid == batch_ref[...]).astype(jnp.float32)
        sums = jnp.dot(oh, h3, preferred_element_type=jnp.float32)
        cnt = jnp.sum(oh, axis=1, keepdims=True)
        pooled = sums / jnp.maximum(cnt, 1.0)
        z = jnp.maximum(jnp.dot(pooled, wf1_ref[...],
                                preferred_element_type=jnp.float32)
                        + bf1_ref[...], 0.0)
        out_ref[...] = jnp.dot(z, wf2_ref[...],
                               preferred_element_type=jnp.float32) + bf2_ref[...]
    return body


def kernel(x, edge_index, batch, W1, b1, W2, b2, W3, b3, Wf1, bf1, Wf2, bf2):
    n, feat = x.shape
    e = edge_index.shape[1]
    et = e + n
    gran = NW * CHUNK
    ep = -(-et // gran) * gran
    cpt = ep // (NW * CHUNK)
    n_pad = -(-(n + 1) // (NS * 128)) * (NS * 128)

    f32 = jnp.float32
    loop = jnp.arange(n, dtype=jnp.int32)
    pad_s = jnp.zeros((ep - et,), jnp.int32)
    pad_d = jnp.full((ep - et,), n, jnp.int32)
    src = jnp.concatenate([edge_index[0], loop, pad_s,
                           jnp.zeros((CHUNK,), jnp.int32)])
    dst = jnp.concatenate([edge_index[1], loop, pad_d,
                           jnp.full((CHUNK,), n, jnp.int32)])

    degp = _sc_degree(n_pad, ep)(dst)
    deg0 = degp[0, :n].reshape(n, 1)
    deg1 = degp[1, :n].reshape(n, 1)

    d1 = W1.shape[1]
    h1, dis = pl.pallas_call(
        _tc_first(n),
        out_shape=(jax.ShapeDtypeStruct((n, d1), f32),
                   jax.ShapeDtypeStruct((n, 1), f32)),
    )(deg0, deg1, x, W1)

    s1 = _sc_scatter(n_pad, ep, d1)(h1, src, dst)

    d2 = W2.shape[1]
    h2 = pl.pallas_call(
        _tc_mid(n, d1), out_shape=jax.ShapeDtypeStruct((n, d2), f32),
    )(s1, dis, b1.reshape(1, -1), W2)

    s2 = _sc_scatter(n_pad, ep, d2)(h2, src, dst)

    d3 = W3.shape[1]
    h3 = pl.pallas_call(
        _tc_mid(n, d2), out_shape=jax.ShapeDtypeStruct((n, d3), f32),
    )(s2, dis, b2.reshape(1, -1), W3)

    s3 = _sc_scatter(n_pad, ep, d3)(h3, src, dst)

    out = pl.pallas_call(
        _tc_head(n), out_shape=jax.ShapeDtypeStruct((G, 1), f32),
    )(s3, dis, b3.reshape(1, -1), batch, Wf1, bf1.reshape(1, -1),
      Wf2, bf2.reshape(1, -1))
    return out


# speedup vs baseline: 1.6715x; 1.0955x over previous
import functools

import jax
import jax.numpy as jnp
from jax import lax
from jax.experimental import pallas as pl
from jax.experimental.pallas import tpu as pltpu
from jax.experimental.pallas import tpu_sc as plsc

NC = 2
NS = 16
NW = NC * NS
LANES = 16
CHUNK = 128
DPAD = 128
G = 64


def _mesh():
    return plsc.VectorSubcoreMesh(core_axis_name="c", subcore_axis_name="s")


def _sc_degree(n_pad, ep):
    ept = ep // NW
    rpt = n_pad // NS

    @functools.partial(
        pl.kernel,
        out_type=jax.ShapeDtypeStruct((NC, n_pad), jnp.float32),
        mesh=_mesh(),
        scratch_types=[
            pltpu.VMEM((ept,), jnp.int32),
            pltpu.VMEM((n_pad,), jnp.float32),
            pltpu.VMEM((rpt,), jnp.float32),
            pltpu.VMEM((rpt,), jnp.float32),
            pltpu.VMEM_SHARED((NS, n_pad), jnp.float32),
        ],
        compiler_params=pltpu.CompilerParams(needs_layout_passes=False),
    )
    def k(dst_hbm, out_hbm, dstbuf, hist, asum, tmp, stage):
        cid = lax.axis_index("c")
        sid = lax.axis_index("s")
        wid = sid * NC + cid

        def zero(i, carry):
            hist[pl.ds(i * LANES, LANES)] = jnp.zeros((LANES,), jnp.float32)
            return carry

        lax.fori_loop(0, n_pad // LANES, zero, 0)
        pltpu.sync_copy(dst_hbm.at[pl.ds(wid * ept, ept)], dstbuf)

        ones16 = jnp.ones((LANES,), jnp.float32)

        def scat(i, carry):
            idxv = dstbuf[pl.ds(i * LANES, LANES)]
            plsc.addupdate_scatter(hist, [idxv], ones16)
            return carry

        lax.fori_loop(0, ept // LANES, scat, 0)

        pltpu.sync_copy(hist, stage.at[sid])
        plsc.subcore_barrier()

        base = sid * rpt
        pltpu.sync_copy(stage.at[0, pl.ds(base, rpt)], asum)
        for t in range(1, NS):
            pltpu.sync_copy(stage.at[t, pl.ds(base, rpt)], tmp)

            def acc(j, carry):
                sl = pl.ds(j * LANES, LANES)
                asum[sl] = asum[sl] + tmp[sl]
                return carry

            lax.fori_loop(0, rpt // LANES, acc, 0)
        pltpu.sync_copy(asum, out_hbm.at[cid, pl.ds(base, rpt)])

    return k


def _sc_scatter(n_pad, ep, d):
    cpt = ep // (NW * CHUNK)
    rpt = n_pad // NS

    @functools.partial(
        pl.kernel,
        out_type=jax.ShapeDtypeStruct((NC, n_pad, d), jnp.float32),
        mesh=_mesh(),
        scratch_types=[
            pltpu.VMEM((2, CHUNK), jnp.int32),
            pltpu.VMEM((CHUNK, d), jnp.float32),
            pltpu.VMEM_SHARED((n_pad, d), jnp.float32),
            pltpu.SemaphoreType.DMA,
        ],
        compiler_params=pltpu.CompilerParams(needs_layout_passes=False,
                                             use_tc_tiling_on_sc=False),
    )
    def k(h_hbm, sd_hbm, out_hbm, idx, rows, acc, sem):
        cid = lax.axis_index("c")
        sid = lax.axis_index("s")
        wid = sid * NC + cid

        def zr(r, carry):
            for j in range(d // LANES):
                rows[r, pl.ds(j * LANES, LANES)] = jnp.zeros((LANES,),
                                                             jnp.float32)
            return carry

        lax.fori_loop(0, 128, zr, 0)
        for z in range(rpt // 128):
            pltpu.sync_copy(rows.at[pl.ds(0, 128)],
                            acc.at[pl.ds(sid * rpt + z * 128, 128)])
        plsc.subcore_barrier()

        def body(c, carry):
            pltpu.sync_copy(sd_hbm.at[wid * cpt + c], idx)
            pltpu.async_copy(h_hbm.at[idx.at[0]], rows, sem).wait()
            pltpu.sync_copy(rows, acc.at[idx.at[1]], add=True)
            return carry

        lax.fori_loop(0, cpt, body, 0)
        plsc.subcore_barrier()
        pltpu.sync_copy(acc.at[pl.ds(sid * rpt, rpt)],
                        out_hbm.at[cid, pl.ds(sid * rpt, rpt)])

    return k


def _tc_first(n):
    def body(d0_ref, dg1_ref, x_ref, w_ref, h_ref, dis_ref):
        dis = lax.rsqrt(d0_ref[...] + dg1_ref[...])
        dis_ref[...] = dis
        h_ref[...] = jnp.dot(x_ref[...], w_ref[...],
                             preferred_element_type=jnp.float32) * dis
    return body


def _tc_mid(n, din):
    def body(s_ref, dis_ref, b_ref, w_ref, out_ref):
        dis = dis_ref[...]
        s = s_ref[0, :n, :] + s_ref[1, :n, :]
        h = jnp.maximum(s * dis + b_ref[...], 0.0)
        out_ref[...] = jnp.dot(h, w_ref[...],
                               preferred_element_type=jnp.float32) * dis
    return body


def _tc_head(n):
    def body(s_ref, dis_ref, b_ref, batch_ref, wf1_ref, bf1_ref,
             wf2_ref, bf2_ref, out_ref):
        s = s_ref[0, :n, :] + s_ref[1, :n, :]
        h3 = jnp.maximum(s * dis_ref[...] + b_ref[...], 0.0)
        gid = lax.broadcasted_iota(jnp.int32, (G, 1), 0)
        oh = (gid == batch_ref[...]).astype(jnp.float32)
        sums = jnp.dot(oh, h3, preferred_element_type=jnp.float32)
        cnt = jnp.sum(oh, axis=1, keepdims=True)
        pooled = sums / jnp.maximum(cnt, 1.0)
        z = jnp.maximum(jnp.dot(pooled, wf1_ref[...],
                                preferred_element_type=jnp.float32)
                        + bf1_ref[...], 0.0)
        out_ref[...] = jnp.dot(z, wf2_ref[...],
                               preferred_element_type=jnp.float32) + bf2_ref[...]
    return body


def kernel(x, edge_index, batch, W1, b1, W2, b2, W3, b3, Wf1, bf1, Wf2, bf2):
    n, feat = x.shape
    e = edge_index.shape[1]
    et = e + n
    gran = NW * CHUNK
    ep = -(-et // gran) * gran
    cpt = ep // (NW * CHUNK)
    n_pad = -(-(n + 1) // (NS * 128)) * (NS * 128)

    f32 = jnp.float32
    loop = jnp.arange(n, dtype=jnp.int32)
    pad_s = jnp.zeros((ep - et,), jnp.int32)
    pad_d = jnp.full((ep - et,), n, jnp.int32)
    src = jnp.concatenate([edge_index[0], loop, pad_s])
    dst = jnp.concatenate([edge_index[1], loop, pad_d])
    sd = jnp.stack([src.reshape(-1, CHUNK), dst.reshape(-1, CHUNK)], axis=1)

    degp = _sc_degree(n_pad, ep)(dst)
    deg0 = degp[0, :n].reshape(n, 1)
    deg1 = degp[1, :n].reshape(n, 1)

    d1 = W1.shape[1]
    h1, dis = pl.pallas_call(
        _tc_first(n),
        out_shape=(jax.ShapeDtypeStruct((n, d1), f32),
                   jax.ShapeDtypeStruct((n, 1), f32)),
    )(deg0, deg1, x, W1)

    s1 = _sc_scatter(n_pad, ep, d1)(h1, sd)

    d2 = W2.shape[1]
    h2 = pl.pallas_call(
        _tc_mid(n, d1), out_shape=jax.ShapeDtypeStruct((n, d2), f32),
    )(s1, dis, b1.reshape(1, -1), W2)

    s2 = _sc_scatter(n_pad, ep, d2)(h2, sd)

    d3 = W3.shape[1]
    h3 = pl.pallas_call(
        _tc_mid(n, d2), out_shape=jax.ShapeDtypeStruct((n, d3), f32),
    )(s2, dis, b2.reshape(1, -1), W3)

    s3 = _sc_scatter(n_pad, ep, d3)(h3, sd)

    out = pl.pallas_call(
        _tc_head(n), out_shape=jax.ShapeDtypeStruct((G, 1), f32),
    )(s3, dis, b3.reshape(1, -1), batch, Wf1, bf1.reshape(1, -1),
      Wf2, bf2.reshape(1, -1))
    return out


# speedup vs baseline: 1.8827x; 1.1264x over previous
import functools

import jax
import jax.numpy as jnp
from jax import lax
from jax.experimental import pallas as pl
from jax.experimental.pallas import tpu as pltpu
from jax.experimental.pallas import tpu_sc as plsc

NC = 2
NS = 16
NW = NC * NS
LANES = 16
CHUNK = 128
DPAD = 128
G = 64


def _mesh():
    return plsc.VectorSubcoreMesh(core_axis_name="c", subcore_axis_name="s")


def _sc_degree(n_pad, ep):
    ept = ep // NW
    rpt = n_pad // NS

    @functools.partial(
        pl.kernel,
        out_type=jax.ShapeDtypeStruct((NC, n_pad), jnp.float32),
        mesh=_mesh(),
        scratch_types=[
            pltpu.VMEM((ept,), jnp.int32),
            pltpu.VMEM((n_pad,), jnp.float32),
            pltpu.VMEM((rpt,), jnp.float32),
            pltpu.VMEM((rpt,), jnp.float32),
            pltpu.VMEM_SHARED((NS, n_pad), jnp.float32),
        ],
        compiler_params=pltpu.CompilerParams(needs_layout_passes=False),
    )
    def k(dst_hbm, out_hbm, dstbuf, hist, asum, tmp, stage):
        cid = lax.axis_index("c")
        sid = lax.axis_index("s")
        wid = sid * NC + cid

        def zero(i, carry):
            hist[pl.ds(i * LANES, LANES)] = jnp.zeros((LANES,), jnp.float32)
            return carry

        lax.fori_loop(0, n_pad // LANES, zero, 0)
        pltpu.sync_copy(dst_hbm.at[pl.ds(wid * ept, ept)], dstbuf)

        ones16 = jnp.ones((LANES,), jnp.float32)

        def scat(i, carry):
            idxv = dstbuf[pl.ds(i * LANES, LANES)]
            plsc.addupdate_scatter(hist, [idxv], ones16)
            return carry

        lax.fori_loop(0, ept // LANES, scat, 0)

        pltpu.sync_copy(hist, stage.at[sid])
        plsc.subcore_barrier()

        base = sid * rpt
        pltpu.sync_copy(stage.at[0, pl.ds(base, rpt)], asum)
        for t in range(1, NS):
            pltpu.sync_copy(stage.at[t, pl.ds(base, rpt)], tmp)

            def acc(j, carry):
                sl = pl.ds(j * LANES, LANES)
                asum[sl] = asum[sl] + tmp[sl]
                return carry

            lax.fori_loop(0, rpt // LANES, acc, 0)
        pltpu.sync_copy(asum, out_hbm.at[cid, pl.ds(base, rpt)])

    return k


def _sc_scatter(n_pad, ep, d):
    cpt = ep // (NW * CHUNK)
    rpt = n_pad // NS

    @functools.partial(
        pl.kernel,
        out_type=jax.ShapeDtypeStruct((NC, n_pad, d), jnp.float32),
        mesh=_mesh(),
        scratch_types=[
            pltpu.VMEM((ep // (NW * CHUNK), 2, CHUNK), jnp.int32),
            pltpu.VMEM((CHUNK, d), jnp.float32),
            pltpu.VMEM_SHARED((n_pad, d), jnp.float32),
            pltpu.SemaphoreType.DMA,
        ],
        compiler_params=pltpu.CompilerParams(needs_layout_passes=False,
                                             use_tc_tiling_on_sc=False),
    )
    def k(h_hbm, sd_hbm, out_hbm, idx, rows, acc, sem):
        cid = lax.axis_index("c")
        sid = lax.axis_index("s")
        wid = sid * NC + cid

        def zr(r, carry):
            for j in range(d // LANES):
                rows[r, pl.ds(j * LANES, LANES)] = jnp.zeros((LANES,),
                                                             jnp.float32)
            return carry

        lax.fori_loop(0, 128, zr, 0)
        for z in range(rpt // 128):
            pltpu.sync_copy(rows.at[pl.ds(0, 128)],
                            acc.at[pl.ds(sid * rpt + z * 128, 128)])
        plsc.subcore_barrier()

        pltpu.sync_copy(sd_hbm.at[pl.ds(wid * cpt, cpt)], idx)

        def body(c, carry):
            pltpu.async_copy(h_hbm.at[idx.at[c, 0]], rows, sem).wait()
            pltpu.sync_copy(rows, acc.at[idx.at[c, 1]], add=True)
            return carry

        lax.fori_loop(0, cpt, body, 0)
        plsc.subcore_barrier()
        pltpu.sync_copy(acc.at[pl.ds(sid * rpt, rpt)],
                        out_hbm.at[cid, pl.ds(sid * rpt, rpt)])

    return k


def _tc_first(n):
    def body(d0_ref, dg1_ref, x_ref, w_ref, h_ref, dis_ref):
        dis = lax.rsqrt(d0_ref[...] + dg1_ref[...])
        dis_ref[...] = dis
        h_ref[...] = jnp.dot(x_ref[...], w_ref[...],
                             preferred_element_type=jnp.float32) * dis
    return body


def _tc_mid(n, din):
    def body(s_ref, dis_ref, b_ref, w_ref, out_ref):
        dis = dis_ref[...]
        s = s_ref[0, :n, :] + s_ref[1, :n, :]
        h = jnp.maximum(s * dis + b_ref[...], 0.0)
        out_ref[...] = jnp.dot(h, w_ref[...],
                               preferred_element_type=jnp.float32) * dis
    return body


def _tc_head(n):
    def body(s_ref, dis_ref, b_ref, batch_ref, wf1_ref, bf1_ref,
             wf2_ref, bf2_ref, out_ref):
        s = s_ref[0, :n, :] + s_ref[1, :n, :]
        h3 = jnp.maximum(s * dis_ref[...] + b_ref[...], 0.0)
        gid = lax.broadcasted_iota(jnp.int32, (G, 1), 0)
        oh = (gid == batch_ref[...]).astype(jnp.float32)
        sums = jnp.dot(oh, h3, preferred_element_type=jnp.float32)
        cnt = jnp.sum(oh, axis=1, keepdims=True)
        pooled = sums / jnp.maximum(cnt, 1.0)
        z = jnp.maximum(jnp.dot(pooled, wf1_ref[...],
                                preferred_element_type=jnp.float32)
                        + bf1_ref[...], 0.0)
        out_ref[...] = jnp.dot(z, wf2_ref[...],
                               preferred_element_type=jnp.float32) + bf2_ref[...]
    return body


def kernel(x, edge_index, batch, W1, b1, W2, b2, W3, b3, Wf1, bf1, Wf2, bf2):
    n, feat = x.shape
    e = edge_index.shape[1]
    et = e + n
    gran = NW * CHUNK
    ep = -(-et // gran) * gran
    cpt = ep // (NW * CHUNK)
    n_pad = -(-(n + 1) // (NS * 128)) * (NS * 128)

    f32 = jnp.float32
    loop = jnp.arange(n, dtype=jnp.int32)
    pad_s = jnp.zeros((ep - et,), jnp.int32)
    pad_d = jnp.full((ep - et,), n, jnp.int32)
    src = jnp.concatenate([edge_index[0], loop, pad_s])
    dst = jnp.concatenate([edge_index[1], loop, pad_d])
    sd = jnp.stack([src.reshape(-1, CHUNK), dst.reshape(-1, CHUNK)], axis=1)

    degp = _sc_degree(n_pad, ep)(dst)
    deg0 = degp[0, :n].reshape(n, 1)
    deg1 = degp[1, :n].reshape(n, 1)

    d1 = W1.shape[1]
    h1, dis = pl.pallas_call(
        _tc_first(n),
        out_shape=(jax.ShapeDtypeStruct((n, d1), f32),
                   jax.ShapeDtypeStruct((n, 1), f32)),
    )(deg0, deg1, x, W1)

    s1 = _sc_scatter(n_pad, ep, d1)(h1, sd)

    d2 = W2.shape[1]
    h2 = pl.pallas_call(
        _tc_mid(n, d1), out_shape=jax.ShapeDtypeStruct((n, d2), f32),
    )(s1, dis, b1.reshape(1, -1), W2)

    s2 = _sc_scatter(n_pad, ep, d2)(h2, sd)

    d3 = W3.shape[1]
    h3 = pl.pallas_call(
        _tc_mid(n, d2), out_shape=jax.ShapeDtypeStruct((n, d3), f32),
    )(s2, dis, b2.reshape(1, -1), W3)

    s3 = _sc_scatter(n_pad, ep, d3)(h3, sd)

    out = pl.pallas_call(
        _tc_head(n), out_shape=jax.ShapeDtypeStruct((G, 1), f32),
    )(s3, dis, b3.reshape(1, -1), batch, Wf1, bf1.reshape(1, -1),
      Wf2, bf2.reshape(1, -1))
    return out


# speedup vs baseline: 1.8837x; 1.0005x over previous
import functools

import jax
import jax.numpy as jnp
from jax import lax
from jax.experimental import pallas as pl
from jax.experimental.pallas import tpu as pltpu
from jax.experimental.pallas import tpu_sc as plsc

NC = 2
NS = 16
NW = NC * NS
LANES = 16
CHUNK = 128
DPAD = 128
G = 64


def _mesh():
    return plsc.VectorSubcoreMesh(core_axis_name="c", subcore_axis_name="s")


def _sc_degree(n_pad, ep):
    ept = ep // NW
    rpt = n_pad // NS

    @functools.partial(
        pl.kernel,
        out_type=jax.ShapeDtypeStruct((NC, n_pad), jnp.float32),
        mesh=_mesh(),
        scratch_types=[
            pltpu.VMEM((ept,), jnp.int32),
            pltpu.VMEM((n_pad,), jnp.float32),
            pltpu.VMEM((rpt,), jnp.float32),
            pltpu.VMEM((rpt,), jnp.float32),
            pltpu.VMEM_SHARED((NS, n_pad), jnp.float32),
        ],
        compiler_params=pltpu.CompilerParams(needs_layout_passes=False),
    )
    def k(dst_hbm, out_hbm, dstbuf, hist, asum, tmp, stage):
        cid = lax.axis_index("c")
        sid = lax.axis_index("s")
        wid = sid * NC + cid

        def zero(i, carry):
            hist[pl.ds(i * LANES, LANES)] = jnp.zeros((LANES,), jnp.float32)
            return carry

        lax.fori_loop(0, n_pad // LANES, zero, 0)
        pltpu.sync_copy(dst_hbm.at[pl.ds(wid * ept, ept)], dstbuf)

        ones16 = jnp.ones((LANES,), jnp.float32)

        def scat(i, carry):
            idxv = dstbuf[pl.ds(i * LANES, LANES)]
            plsc.addupdate_scatter(hist, [idxv], ones16)
            return carry

        lax.fori_loop(0, ept // LANES, scat, 0)

        pltpu.sync_copy(hist, stage.at[sid])
        plsc.subcore_barrier()

        base = sid * rpt
        pltpu.sync_copy(stage.at[0, pl.ds(base, rpt)], asum)
        for t in range(1, NS):
            pltpu.sync_copy(stage.at[t, pl.ds(base, rpt)], tmp)

            def acc(j, carry):
                sl = pl.ds(j * LANES, LANES)
                asum[sl] = asum[sl] + tmp[sl]
                return carry

            lax.fori_loop(0, rpt // LANES, acc, 0)
        pltpu.sync_copy(asum, out_hbm.at[cid, pl.ds(base, rpt)])

    return k


def _sc_scatter(n_pad, ep, d):
    cpt = ep // (NW * CHUNK)
    rpt = n_pad // NS

    @functools.partial(
        pl.kernel,
        out_type=jax.ShapeDtypeStruct((NC, n_pad, d), jnp.float32),
        mesh=_mesh(),
        scratch_types=[
            pltpu.VMEM((ep // (NW * CHUNK), 2, CHUNK), jnp.int32),
            pltpu.VMEM((CHUNK, d), jnp.float32),
            pltpu.VMEM_SHARED((n_pad, d), jnp.float32),
            pltpu.SemaphoreType.DMA,
        ],
        compiler_params=pltpu.CompilerParams(needs_layout_passes=False,
                                             use_tc_tiling_on_sc=False),
    )
    def k(h_hbm, sd_hbm, out_hbm, idx, rows, acc, sem):
        cid = lax.axis_index("c")
        sid = lax.axis_index("s")
        wid = sid * NC + cid

        def zr(r, carry):
            for j in range(d // LANES):
                rows[r, pl.ds(j * LANES, LANES)] = jnp.zeros((LANES,),
                                                             jnp.float32)
            return carry

        lax.fori_loop(0, 128, zr, 0)
        for z in range(rpt // 128):
            pltpu.sync_copy(rows.at[pl.ds(0, 128)],
                            acc.at[pl.ds(sid * rpt + z * 128, 128)])
        plsc.subcore_barrier()

        pltpu.sync_copy(sd_hbm.at[pl.ds(wid * cpt, cpt)], idx)

        def body(c, carry):
            pltpu.async_copy(h_hbm.at[idx.at[c, 0]], rows, sem).wait()
            pltpu.sync_copy(rows, acc.at[idx.at[c, 1]], add=True)
            return carry

        lax.fori_loop(0, cpt, body, 0)
        plsc.subcore_barrier()
        pltpu.sync_copy(acc.at[pl.ds(sid * rpt, rpt)],
                        out_hbm.at[cid, pl.ds(sid * rpt, rpt)])

    return k


def _tc_first(n):
    def body(d0_ref, dg1_ref, x_ref, w_ref, h_ref, dis_ref):
        dis = 1.0 / jnp.sqrt(d0_ref[...] + dg1_ref[...])
        dis_ref[...] = dis
        h_ref[...] = jnp.dot(x_ref[...], w_ref[...],
                             preferred_element_type=jnp.float32) * dis
    return body


def _tc_mid(n, din):
    def body(s_ref, dis_ref, b_ref, w_ref, out_ref):
        dis = dis_ref[...]
        s = s_ref[0, :n, :] + s_ref[1, :n, :]
        h = jnp.maximum(s * dis + b_ref[...], 0.0)
        out_ref[...] = jnp.dot(h, w_ref[...],
                               preferred_element_type=jnp.float32) * dis
    return body


def _tc_head(n):
    def body(s_ref, dis_ref, b_ref, batch_ref, wf1_ref, bf1_ref,
             wf2_ref, bf2_ref, out_ref):
        s = s_ref[0, :n, :] + s_ref[1, :n, :]
        h3 = jnp.maximum(s * dis_ref[...] + b_ref[...], 0.0)
        gid = lax.broadcasted_iota(jnp.int32, (G, 1), 0)
        oh = (gid == batch_ref[...]).astype(jnp.float32)
        sums = jnp.dot(oh, h3, preferred_element_type=jnp.float32)
        cnt = jnp.sum(oh, axis=1, keepdims=True)
        pooled = sums / jnp.maximum(cnt, 1.0)
        z = jnp.maximum(jnp.dot(pooled, wf1_ref[...],
                                preferred_element_type=jnp.float32)
                        + bf1_ref[...], 0.0)
        out_ref[...] = jnp.dot(z, wf2_ref[...],
                               preferred_element_type=jnp.float32) + bf2_ref[...]
    return body


def kernel(x, edge_index, batch, W1, b1, W2, b2, W3, b3, Wf1, bf1, Wf2, bf2):
    n, feat = x.shape
    e = edge_index.shape[1]
    et = e + n
    gran = NW * CHUNK
    ep = -(-et // gran) * gran
    cpt = ep // (NW * CHUNK)
    n_pad = -(-(n + 1) // (NS * 128)) * (NS * 128)

    f32 = jnp.float32
    loop = jnp.arange(n, dtype=jnp.int32)
    pad_s = jnp.zeros((ep - et,), jnp.int32)
    pad_d = jnp.full((ep - et,), n, jnp.int32)
    src = jnp.concatenate([edge_index[0], loop, pad_s])
    dst = jnp.concatenate([edge_index[1], loop, pad_d])
    sd = jnp.stack([src.reshape(-1, CHUNK), dst.reshape(-1, CHUNK)], axis=1)

    degp = _sc_degree(n_pad, ep)(dst)
    deg0 = degp[0, :n].reshape(n, 1)
    deg1 = degp[1, :n].reshape(n, 1)

    d1 = W1.shape[1]
    h1, dis = pl.pallas_call(
        _tc_first(n),
        out_shape=(jax.ShapeDtypeStruct((n, d1), f32),
                   jax.ShapeDtypeStruct((n, 1), f32)),
    )(deg0, deg1, x, W1)

    s1 = _sc_scatter(n_pad, ep, d1)(h1, sd)

    d2 = W2.shape[1]
    h2 = pl.pallas_call(
        _tc_mid(n, d1), out_shape=jax.ShapeDtypeStruct((n, d2), f32),
    )(s1, dis, b1.reshape(1, -1), W2)

    s2 = _sc_scatter(n_pad, ep, d2)(h2, sd)

    d3 = W3.shape[1]
    h3 = pl.pallas_call(
        _tc_mid(n, d2), out_shape=jax.ShapeDtypeStruct((n, d3), f32),
    )(s2, dis, b2.reshape(1, -1), W3)

    s3 = _sc_scatter(n_pad, ep, d3)(h3, sd)

    out = pl.pallas_call(
        _tc_head(n), out_shape=jax.ShapeDtypeStruct((G, 1), f32),
    )(s3, dis, b3.reshape(1, -1), batch, Wf1, bf1.reshape(1, -1),
      Wf2, bf2.reshape(1, -1))
    return out


# speedup vs baseline: 2.1775x; 1.1560x over previous
import functools

import jax
import jax.numpy as jnp
from jax import lax
from jax.experimental import pallas as pl
from jax.experimental.pallas import tpu as pltpu
from jax.experimental.pallas import tpu_sc as plsc

NC = 2
NS = 16
NW = NC * NS
LANES = 16
CHUNK = 128
DPAD = 128
G = 64
BLK = 27


def _mesh():
    return plsc.VectorSubcoreMesh(core_axis_name="c", subcore_axis_name="s")


def _sc_degree(n_pad, ep):
    ept = ep // NW
    rpt = n_pad // NS

    @functools.partial(
        pl.kernel,
        out_type=jax.ShapeDtypeStruct((NC, n_pad), jnp.float32),
        mesh=_mesh(),
        scratch_types=[
            pltpu.VMEM((ept,), jnp.int32),
            pltpu.VMEM((n_pad,), jnp.float32),
            pltpu.VMEM((rpt,), jnp.float32),
            pltpu.VMEM((rpt,), jnp.float32),
            pltpu.VMEM_SHARED((NS, n_pad), jnp.float32),
        ],
        compiler_params=pltpu.CompilerParams(needs_layout_passes=False),
    )
    def k(dst_hbm, out_hbm, dstbuf, hist, asum, tmp, stage):
        cid = lax.axis_index("c")
        sid = lax.axis_index("s")
        wid = sid * NC + cid

        def zero(i, carry):
            hist[pl.ds(i * LANES, LANES)] = jnp.zeros((LANES,), jnp.float32)
            return carry

        lax.fori_loop(0, n_pad // LANES, zero, 0)
        pltpu.sync_copy(dst_hbm.at[pl.ds(wid * ept, ept)], dstbuf)

        ones16 = jnp.ones((LANES,), jnp.float32)

        def scat(i, carry):
            idxv = dstbuf[pl.ds(i * LANES, LANES)]
            plsc.addupdate_scatter(hist, [idxv], ones16)
            return carry

        lax.fori_loop(0, ept // LANES, scat, 0)

        pltpu.sync_copy(hist, stage.at[sid])
        plsc.subcore_barrier()

        base = sid * rpt
        pltpu.sync_copy(stage.at[0, pl.ds(base, rpt)], asum)
        for t in range(1, NS):
            pltpu.sync_copy(stage.at[t, pl.ds(base, rpt)], tmp)

            def acc(j, carry):
                sl = pl.ds(j * LANES, LANES)
                asum[sl] = asum[sl] + tmp[sl]
                return carry

            lax.fori_loop(0, rpt // LANES, acc, 0)
        pltpu.sync_copy(asum, out_hbm.at[cid, pl.ds(base, rpt)])

    return k


def _sc_scatter(n_pad, ep, d):
    cpt = ep // (NW * CHUNK)
    rpt = n_pad // NS

    @functools.partial(
        pl.kernel,
        out_type=jax.ShapeDtypeStruct((NC, n_pad, d), jnp.float32),
        mesh=_mesh(),
        scratch_types=[
            pltpu.VMEM((BLK, 2, CHUNK), jnp.int32),
            pltpu.VMEM((2, CHUNK, d), jnp.float32),
            pltpu.VMEM_SHARED((n_pad, d), jnp.float32),
            pltpu.SemaphoreType.DMA,
        ],
        compiler_params=pltpu.CompilerParams(needs_layout_passes=False,
                                             use_tc_tiling_on_sc=False),
    )
    def k(h_hbm, sd_hbm, out_hbm, idx, rows, acc, sem):
        cid = lax.axis_index("c")
        sid = lax.axis_index("s")
        wid = sid * NC + cid

        def zr(r, carry):
            for j in range(d // LANES):
                rows[0, r, pl.ds(j * LANES, LANES)] = jnp.zeros((LANES,),
                                                                jnp.float32)
            return carry

        lax.fori_loop(0, 128, zr, 0)
        for z in range(rpt // 128):
            pltpu.sync_copy(rows.at[0],
                            acc.at[pl.ds(sid * rpt + z * 128, 128)])

        plsc.subcore_barrier()

        def body(c, carry):
            b = lax.rem(c, 2)
            pltpu.make_async_copy(h_hbm.at[idx.at[c, 0]], rows.at[b],
                                  sem).wait()
            pltpu.async_copy(h_hbm.at[idx.at[c + 1, 0]], rows.at[1 - b], sem)
            pltpu.sync_copy(rows.at[b], acc.at[idx.at[c, 1]], add=True)
            return carry

        for blk in range(cpt // BLK):
            pltpu.sync_copy(sd_hbm.at[pl.ds(wid * cpt + blk * BLK, BLK)], idx)
            pltpu.async_copy(h_hbm.at[idx.at[0, 0]], rows.at[0], sem)
            lax.fori_loop(0, BLK - 1, body, 0)
            bl = (BLK - 1) % 2
            pltpu.make_async_copy(h_hbm.at[idx.at[BLK - 1, 0]], rows.at[bl],
                                  sem).wait()
            pltpu.sync_copy(rows.at[bl], acc.at[idx.at[BLK - 1, 1]], add=True)
        plsc.subcore_barrier()
        pltpu.sync_copy(acc.at[pl.ds(sid * rpt, rpt)],
                        out_hbm.at[cid, pl.ds(sid * rpt, rpt)])

    return k


def _tc_first(n):
    def body(d0_ref, dg1_ref, x_ref, w_ref, h_ref, dis_ref):
        dis = 1.0 / jnp.sqrt(d0_ref[...] + dg1_ref[...])
        dis_ref[...] = dis
        h_ref[...] = jnp.dot(x_ref[...], w_ref[...],
                             preferred_element_type=jnp.float32) * dis
    return body


def _tc_mid(n, din):
    def body(s_ref, dis_ref, b_ref, w_ref, out_ref):
        dis = dis_ref[...]
        s = s_ref[0, :n, :] + s_ref[1, :n, :]
        h = jnp.maximum(s * dis + b_ref[...], 0.0)
        out_ref[...] = jnp.dot(h, w_ref[...],
                               preferred_element_type=jnp.float32) * dis
    return body


def _tc_head(n):
    def body(s_ref, dis_ref, b_ref, batch_ref, wf1_ref, bf1_ref,
             wf2_ref, bf2_ref, out_ref):
        s = s_ref[0, :n, :] + s_ref[1, :n, :]
        h3 = jnp.maximum(s * dis_ref[...] + b_ref[...], 0.0)
        gid = lax.broadcasted_iota(jnp.int32, (G, 1), 0)
        oh = (gid == batch_ref[...]).astype(jnp.float32)
        sums = jnp.dot(oh, h3, preferred_element_type=jnp.float32)
        cnt = jnp.sum(oh, axis=1, keepdims=True)
        pooled = sums / jnp.maximum(cnt, 1.0)
        z = jnp.maximum(jnp.dot(pooled, wf1_ref[...],
                                preferred_element_type=jnp.float32)
                        + bf1_ref[...], 0.0)
        out_ref[...] = jnp.dot(z, wf2_ref[...],
                               preferred_element_type=jnp.float32) + bf2_ref[...]
    return body


def kernel(x, edge_index, batch, W1, b1, W2, b2, W3, b3, Wf1, bf1, Wf2, bf2):
    n, feat = x.shape
    e = edge_index.shape[1]
    et = e + n
    gran = NW * CHUNK * BLK
    ep = -(-et // gran) * gran
    cpt = ep // (NW * CHUNK)
    n_pad = -(-(n + 1) // (NS * 128)) * (NS * 128)

    f32 = jnp.float32
    loop = jnp.arange(n, dtype=jnp.int32)
    pad_s = jnp.zeros((ep - et,), jnp.int32)
    pad_d = jnp.full((ep - et,), n, jnp.int32)
    src = jnp.concatenate([edge_index[0], loop, pad_s])
    dst = jnp.concatenate([edge_index[1], loop, pad_d])
    sd = jnp.stack([src.reshape(-1, CHUNK), dst.reshape(-1, CHUNK)], axis=1)

    degp = _sc_degree(n_pad, ep)(dst)
    deg0 = degp[0, :n].reshape(n, 1)
    deg1 = degp[1, :n].reshape(n, 1)

    d1 = W1.shape[1]
    h1, dis = pl.pallas_call(
        _tc_first(n),
        out_shape=(jax.ShapeDtypeStruct((n, d1), f32),
                   jax.ShapeDtypeStruct((n, 1), f32)),
    )(deg0, deg1, x, W1)

    s1 = _sc_scatter(n_pad, ep, d1)(h1, sd)

    d2 = W2.shape[1]
    h2 = pl.pallas_call(
        _tc_mid(n, d1), out_shape=jax.ShapeDtypeStruct((n, d2), f32),
    )(s1, dis, b1.reshape(1, -1), W2)

    s2 = _sc_scatter(n_pad, ep, d2)(h2, sd)

    d3 = W3.shape[1]
    h3 = pl.pallas_call(
        _tc_mid(n, d2), out_shape=jax.ShapeDtypeStruct((n, d3), f32),
    )(s2, dis, b2.reshape(1, -1), W3)

    s3 = _sc_scatter(n_pad, ep, d3)(h3, sd)

    out = pl.pallas_call(
        _tc_head(n), out_shape=jax.ShapeDtypeStruct((G, 1), f32),
    )(s3, dis, b3.reshape(1, -1), batch, Wf1, bf1.reshape(1, -1),
      Wf2, bf2.reshape(1, -1))
    return out
